# baseline (speedup 1.0000x reference)
import functools

import jax
import jax.numpy as jnp
from jax import lax
from jax.experimental import pallas as pl
from jax.experimental.pallas import tpu as pltpu
from jax.experimental.pallas import tpu_sc as plsc

N = 10000
E = 160000
D = 256
DH = 128
DQ = 64
NC = 2
NS = 16
CH = 128
K = 80
NB = 3
HALF = 5120
ACC_ROWS = 5248
LIST_LEN = K * CH + CH
ZPT = ACC_ROWS // NS
DPT = HALF // NS
MASK14 = 16383
PAD_A = HALF
PAD_B = 2 * HALF
EPAD = NS * K * CH
EPW = EPAD // (NC * NS)
ROWS_ACC = 10240
TRASH = N
RPT = ROWS_ACC // NS
RB = 1000
GRID = N // RB

@functools.lru_cache(maxsize=1)
def _sc_kernels():
    mesh = plsc.VectorSubcoreMesh(
        core_axis_name="c", subcore_axis_name="s",
        num_cores=NC, num_subcores=NS)

    deg_kernel = functools.partial(
        pl.kernel,
        out_type=jax.ShapeDtypeStruct((NC * NS, ROWS_ACC), jnp.float32),
        mesh=mesh,
        scratch_types=[
            pltpu.VMEM((EPW,), jnp.int32),
            pltpu.VMEM((ROWS_ACC,), jnp.float32),
        ],
        compiler_params=pltpu.CompilerParams(needs_layout_passes=False),
    )(_deg_body)

    scatter_kernel = functools.partial(
        pl.kernel,
        out_type=jax.ShapeDtypeStruct((NC, ROWS_ACC, DH), jnp.float32),
        mesh=mesh,
        scratch_types=[
            pltpu.VMEM((K, CH), jnp.int32),
            pltpu.VMEM((LIST_LEN,), jnp.int32),
            pltpu.VMEM((LIST_LEN,), jnp.int32),
            pltpu.VMEM((NB, CH), jnp.int32),
            pltpu.VMEM((NB, CH), jnp.int32),
            *[pltpu.VMEM((CH, DH), jnp.float32) for _ in range(NB)],
            pltpu.VMEM_SHARED((ACC_ROWS, DH), jnp.float32),
            *[pltpu.SemaphoreType.DMA for _ in range(2 * NB)],
        ],
        compiler_params=pltpu.CompilerParams(needs_layout_passes=False),
    )(_scatter_body)
    return deg_kernel, scatter_kernel


def _deg_body(dst_hbm, out_hbm, dstv, hist):
    cid = lax.axis_index("c")
    sid = lax.axis_index("s")
    wid = sid * NC + cid

    def zb(i, carry):
        hist[pl.ds(i * 16, 16)] = jnp.zeros((16,), jnp.float32)
        return carry

    lax.fori_loop(0, ROWS_ACC // 16, zb, 0)
    pltpu.sync_copy(dst_hbm.at[wid], dstv)
    ones = jnp.ones((16,), jnp.float32)

    def body(j, carry):
        idx = dstv[pl.ds(j * 16, 16)]
        plsc.addupdate_scatter(hist, [idx], ones)
        return carry

    lax.fori_loop(0, EPW // 16, body, 0)
    pltpu.sync_copy(hist, out_hbm.at[wid])


def _scatter_body(g_hbm, pack_hbm, out_hbm,
                  packv, list_a, list_b, gidxc, didxc, *rest):
    rows = rest[:NB]
    acc = rest[NB]
    gsem = rest[NB + 1:NB + 1 + NB]
    ssem = rest[NB + 1 + NB:]
    cid = lax.axis_index("c")
    sid = lax.axis_index("s")

    pltpu.sync_copy(pack_hbm.at[sid], packv)

    def cb(i, carry):
        ca, cbb = carry
        r = i // (CH // 16)
        c = (i % (CH // 16)) * 16
        p = packv[r, pl.ds(c, 16)]
        d = p & MASK14
        m_a = d < HALF
        plsc.store_compressed(list_a.at[pl.ds(ca, 16)], p, mask=m_a)
        plsc.store_compressed(list_b.at[pl.ds(cbb, 16)], p,
                              mask=jnp.logical_not(m_a))
        na = jnp.sum(m_a.astype(jnp.int32))
        return ca + na, cbb + (16 - na)

    cnt_a, cnt_b = lax.fori_loop(
        0, K * (CH // 16), cb, (jnp.int32(0), jnp.int32(0)))

    for t in range(CH // 16):
        list_a[pl.ds(cnt_a + t * 16, 16)] = jnp.full((16,), PAD_A, jnp.int32)
        list_b[pl.ds(cnt_b + t * 16, 16)] = jnp.full((16,), PAD_B, jnp.int32)
    nc_a = (cnt_a + CH - 1) // CH
    nc_b = (cnt_b + CH - 1) // CH

    for half in range(2):
        nc = nc_a if half == 0 else nc_b
        lst = list_a if half == 0 else list_b
        base = half * HALF

        def zb(i, carry):
            r = i // (DH // 16)
            c = (i % (DH // 16)) * 16
            rows[0][r, pl.ds(c, 16)] = jnp.zeros((16,), jnp.float32)
            return carry

        lax.fori_loop(0, CH * (DH // 16), zb, 0)
        pltpu.sync_copy(rows[0], acc.at[pl.ds(sid * ZPT, CH)])
        pltpu.sync_copy(rows[0], acc.at[pl.ds(sid * ZPT + CH, CH)])
        pltpu.sync_copy(rows[0].at[pl.ds(0, ZPT - 2 * CH)],
                        acc.at[pl.ds(sid * ZPT + 2 * CH, ZPT - 2 * CH)])
        plsc.subcore_barrier()

        def unpack(q, b):
            for t in range(CH // 16):
                p = lst[pl.ds(q * CH + t * 16, 16)]
                src16 = lax.shift_right_logical(p, 14)
                gidxc[b, pl.ds(t * 16, 16)] = src16 * 2 + cid
                didxc[b, pl.ds(t * 16, 16)] = (p & MASK14) - base

        def _gather(b):
            pltpu.async_copy(g_hbm.at[gidxc.at[b]], rows[b], gsem[b])

        def _scatter(b):
            pltpu.async_copy(rows[b], acc.at[didxc.at[b]], ssem[b], add=True)

        def _gwait(b):
            pltpu.make_async_copy(g_hbm.at[gidxc.at[b]], rows[b],
                                  gsem[b]).wait()

        def _swait(b):
            pltpu.make_async_copy(rows[b], acc.at[didxc.at[b]],
                                  ssem[b]).wait()

        def grp(g, carry):
            for b in range(NB):
                qp = g * NB + b - NB

                @pl.when((qp >= 0) & (qp < nc))
                def _():
                    _gwait(b)
                    _scatter(b)

            for b in range(NB):
                q = g * NB + b
                qp = q - NB

                @pl.when((qp >= 0) & (qp < nc))
                def _():
                    _swait(b)

                @pl.when(q < nc)
                def _():
                    unpack(q, b)
                    _gather(b)
            return carry

        lax.fori_loop(0, (nc + NB - 1) // NB + 1, grp, 0)
        plsc.subcore_barrier()
        pltpu.sync_copy(acc.at[pl.ds(sid * DPT, DPT)],
                        out_hbm.at[cid, pl.ds(base + sid * DPT, DPT)])
        plsc.subcore_barrier()


def _mm_scale_body(deg_ref, x_ref, w_ref, out_ref):
    deg = jnp.sum(deg_ref[...], axis=1) + 1.0
    dinv = lax.rsqrt(deg)
    h = jnp.dot(x_ref[...], w_ref[...], preferred_element_type=jnp.float32)
    out_ref[...] = h * dinv[:, None]


def _combine_mm_body(deg_ref, sp_ref, g_ref, b_ref, w_ref, out_ref):
    deg = jnp.sum(deg_ref[...], axis=1) + 1.0
    dinv = lax.rsqrt(deg)
    sp = sp_ref[...]
    s = jnp.concatenate([sp[0], sp[1]], axis=-1)
    o = (s + g_ref[...]) * dinv[:, None] + b_ref[...]
    h = jnp.dot(o, w_ref[...], preferred_element_type=jnp.float32)
    out_ref[...] = h * dinv[:, None]


def _combine_stats_body(deg_ref, sp_ref, g_ref, b_ref, o_ref, stats_ref):
    i = pl.program_id(0)
    deg = jnp.sum(deg_ref[...], axis=1) + 1.0
    dinv = lax.rsqrt(deg)
    sp = sp_ref[...]
    s = jnp.concatenate([sp[0], sp[1]], axis=-1)
    o = (s + g_ref[...]) * dinv[:, None] + b_ref[...]
    o_ref[...] = o
    blk = jnp.stack([jnp.sum(o, axis=0), jnp.sum(o * o, axis=0)])

    @pl.when(i == 0)
    def _():
        stats_ref[...] = blk

    @pl.when(i > 0)
    def _():
        stats_ref[...] = stats_ref[...] + blk


def _norm_body(o_ref, stats_ref, out_ref):
    st = stats_ref[...]
    mean = st[0]
    nf = jnp.float32(N)
    mu = mean / nf
    var = (st[1] - nf * mu * mu) / (nf - 1.0)
    rstd = lax.rsqrt(var)
    out_ref[...] = (o_ref[...] - mu[None, :]) * rstd[None, :]


def _deg_spec():
    return pl.BlockSpec((RB, NC * NS), lambda i: (i, 0))


def _row_spec():
    return pl.BlockSpec((RB, D), lambda i: (i, 0))


def _sp_spec():
    return pl.BlockSpec((NC, RB, DH), lambda i: (0, i, 0))


def _full_spec(shape):
    return pl.BlockSpec(shape, lambda i: tuple(0 for _ in shape))


def kernel(x, edge_index, W1, b1, W2, b2):
    src = edge_index[0].astype(jnp.int32)
    dst = edge_index[1].astype(jnp.int32)
    pad = EPAD - E
    srcp = jnp.concatenate([src, jnp.zeros((pad,), jnp.int32)])
    dstp = jnp.concatenate([dst, jnp.full((pad,), TRASH, jnp.int32)])
    dst_w = dstp.reshape(NC * NS, EPW)
    pack3 = ((srcp << 14) | dstp).reshape(NS, K, CH)
    b1r = b1.reshape(1, D)
    b2r = b2.reshape(1, D)

    deg_kernel, scatter_kernel = _sc_kernels()
    deg_parts = deg_kernel(dst_w).T

    g1 = pl.pallas_call(
        _mm_scale_body,
        grid=(GRID,),
        in_specs=[_deg_spec(), _row_spec(), _full_spec((D, D))],
        out_specs=_row_spec(),
        out_shape=jax.ShapeDtypeStruct((N, D), jnp.float32),
    )(deg_parts, x, W1)

    s1 = scatter_kernel(g1.reshape(2 * N, DH), pack3)

    g2 = pl.pallas_call(
        _combine_mm_body,
        grid=(GRID,),
        in_specs=[_deg_spec(), _sp_spec(), _row_spec(),
                  _full_spec((1, D)), _full_spec((D, D))],
        out_specs=_row_spec(),
        out_shape=jax.ShapeDtypeStruct((N, D), jnp.float32),
    )(deg_parts, s1, g1, b1r, W2)

    s2 = scatter_kernel(g2.reshape(2 * N, DH), pack3)

    o2, stats = pl.pallas_call(
        _combine_stats_body,
        grid=(GRID,),
        in_specs=[_deg_spec(), _sp_spec(), _row_spec(), _full_spec((1, D))],
        out_specs=[_row_spec(), _full_spec((2, D))],
        out_shape=[jax.ShapeDtypeStruct((N, D), jnp.float32),
                   jax.ShapeDtypeStruct((2, D), jnp.float32)],
    )(deg_parts, s2, g2, b2r)

    out = pl.pallas_call(
        _norm_body,
        grid=(GRID,),
        in_specs=[_row_spec(), _full_spec((2, D))],
        out_specs=_row_spec(),
        out_shape=jax.ShapeDtypeStruct((N, D), jnp.float32),
    )(o2, stats)
    return out


# speedup vs baseline: 1.1378x; 1.1378x over previous
import functools

import jax
import jax.numpy as jnp
from jax import lax
from jax.experimental import pallas as pl
from jax.experimental.pallas import tpu as pltpu
from jax.experimental.pallas import tpu_sc as plsc

N = 10000
E = 160000
D = 256
DH = 128
DQ = 64
NC = 2
NS = 16
CH = 128
K = 80
NB = 5
EPAD = NS * K * CH
EPW = EPAD // (NC * NS)
ROWS_ACC = 10240
TRASH = N
RPT = ROWS_ACC // NS
RB = 1000
GRID = N // RB

@functools.lru_cache(maxsize=1)
def _sc_kernels():
    mesh = plsc.VectorSubcoreMesh(
        core_axis_name="c", subcore_axis_name="s",
        num_cores=NC, num_subcores=NS)

    deg_kernel = functools.partial(
        pl.kernel,
        out_type=jax.ShapeDtypeStruct((NC * NS, ROWS_ACC), jnp.float32),
        mesh=mesh,
        scratch_types=[
            pltpu.VMEM((EPW,), jnp.int32),
            pltpu.VMEM((ROWS_ACC,), jnp.float32),
        ],
        compiler_params=pltpu.CompilerParams(
            needs_layout_passes=False, use_tc_tiling_on_sc=False),
    )(_deg_body)

    scatter_kernel = functools.partial(
        pl.kernel,
        out_type=jax.ShapeDtypeStruct((2 * NC, ROWS_ACC, DQ), jnp.float32),
        mesh=mesh,
        scratch_types=[
            pltpu.VMEM((K, CH), jnp.int32),
            pltpu.VMEM((K, CH), jnp.int32),
            pltpu.VMEM((K, CH), jnp.int32),
            *[pltpu.VMEM((CH, DQ), jnp.float32) for _ in range(NB)],
            pltpu.VMEM((CH, DQ), jnp.float32),
            pltpu.VMEM_SHARED((ROWS_ACC, DQ), jnp.float32),
            *[pltpu.SemaphoreType.DMA for _ in range(2 * NB)],
        ],
        compiler_params=pltpu.CompilerParams(
            needs_layout_passes=False, use_tc_tiling_on_sc=False),
    )(_scatter_body)
    return deg_kernel, scatter_kernel


def _deg_body(dst_hbm, out_hbm, dstv, hist):
    cid = lax.axis_index("c")
    sid = lax.axis_index("s")
    wid = sid * NC + cid

    def zb(i, carry):
        hist[pl.ds(i * 16, 16)] = jnp.zeros((16,), jnp.float32)
        return carry

    lax.fori_loop(0, ROWS_ACC // 16, zb, 0)
    pltpu.sync_copy(dst_hbm.at[wid], dstv)
    ones = jnp.ones((16,), jnp.float32)

    def body(j, carry):
        idx = dstv[pl.ds(j * 16, 16)]
        plsc.addupdate_scatter(hist, [idx], ones)
        return carry

    lax.fori_loop(0, EPW // 16, body, 0)
    pltpu.sync_copy(hist, out_hbm.at[wid])


def _scatter_body(g_hbm, src_hbm, dst_hbm, out_hbm,
                  srcv, gidx, didx, *rest):
    rows = rest[:NB]
    zbuf, acc = rest[NB], rest[NB + 1]
    gsem = rest[NB + 2:NB + 2 + NB]
    ssem = rest[NB + 2 + NB:]
    cid = lax.axis_index("c")
    sid = lax.axis_index("s")

    def zb(i, carry):
        r = i // (DQ // 16)
        c = (i % (DQ // 16)) * 16
        zbuf[r, pl.ds(c, 16)] = jnp.zeros((16,), jnp.float32)
        return carry

    lax.fori_loop(0, CH * (DQ // 16), zb, 0)

    pltpu.sync_copy(src_hbm.at[sid], srcv)
    pltpu.sync_copy(dst_hbm.at[sid], didx)

    for qpass in range(2):
        q = cid * 2 + qpass
        for k in range(RPT // CH):
            pltpu.sync_copy(zbuf, acc.at[pl.ds(sid * RPT + k * CH, CH)])

        def tb(i, carry):
            r = i // (CH // 16)
            c = (i % (CH // 16)) * 16
            s = srcv[r, pl.ds(c, 16)]
            gidx[r, pl.ds(c, 16)] = s * 4 + q
            return carry

        lax.fori_loop(0, K * (CH // 16), tb, 0)
        plsc.subcore_barrier()

        def _gather(j, b):
            pltpu.async_copy(g_hbm.at[gidx.at[j]], rows[b], gsem[b])

        def _scatter(j, b):
            pltpu.async_copy(rows[b], acc.at[didx.at[j]], ssem[b], add=True)

        def _gwait(b):
            pltpu.make_async_copy(g_hbm.at[gidx.at[0]], rows[b],
                                  gsem[b]).wait()

        def _swait(b):
            pltpu.make_async_copy(rows[b], acc.at[didx.at[0]],
                                  ssem[b]).wait()

        for b in range(NB):
            _gather(b, b)

        def grp(g, carry):
            base = g * NB
            for b in range(NB):
                _gwait(b)
                _scatter(base - NB + b, b)
                _swait(b)
                _gather(base + b, b)
            return carry

        lax.fori_loop(1, K // NB, grp, 0)
        for b in range(NB):
            _gwait(b)
            _scatter(K - NB + b, b)
            _swait(b)
        plsc.subcore_barrier()
        pltpu.sync_copy(acc.at[pl.ds(sid * RPT, RPT)],
                        out_hbm.at[q, pl.ds(sid * RPT, RPT)])


def _mm_scale_body(deg_ref, x_ref, w_ref, out_ref):
    deg = jnp.sum(deg_ref[...], axis=1) + 1.0
    dinv = lax.rsqrt(deg)
    h = jnp.dot(x_ref[...], w_ref[...], preferred_element_type=jnp.float32)
    out_ref[...] = h * dinv[:, None]


def _combine_mm_body(deg_ref, sp_ref, g_ref, b_ref, w_ref, out_ref):
    deg = jnp.sum(deg_ref[...], axis=1) + 1.0
    dinv = lax.rsqrt(deg)
    sp = sp_ref[...]
    s = jnp.concatenate([sp[0], sp[1], sp[2], sp[3]], axis=-1)
    o = (s + g_ref[...]) * dinv[:, None] + b_ref[...]
    h = jnp.dot(o, w_ref[...], preferred_element_type=jnp.float32)
    out_ref[...] = h * dinv[:, None]


def _combine_stats_body(deg_ref, sp_ref, g_ref, b_ref, o_ref, stats_ref):
    i = pl.program_id(0)
    deg = jnp.sum(deg_ref[...], axis=1) + 1.0
    dinv = lax.rsqrt(deg)
    sp = sp_ref[...]
    s = jnp.concatenate([sp[0], sp[1], sp[2], sp[3]], axis=-1)
    o = (s + g_ref[...]) * dinv[:, None] + b_ref[...]
    o_ref[...] = o
    blk = jnp.stack([jnp.sum(o, axis=0), jnp.sum(o * o, axis=0)])

    @pl.when(i == 0)
    def _():
        stats_ref[...] = blk

    @pl.when(i > 0)
    def _():
        stats_ref[...] = stats_ref[...] + blk


def _norm_body(o_ref, stats_ref, out_ref):
    st = stats_ref[...]
    mean = st[0]
    nf = jnp.float32(N)
    mu = mean / nf
    var = (st[1] - nf * mu * mu) / (nf - 1.0)
    rstd = lax.rsqrt(var)
    out_ref[...] = (o_ref[...] - mu[None, :]) * rstd[None, :]


def _deg_spec():
    return pl.BlockSpec((RB, NC * NS), lambda i: (i, 0))


def _row_spec():
    return pl.BlockSpec((RB, D), lambda i: (i, 0))


def _sp_spec():
    return pl.BlockSpec((2 * NC, RB, DQ), lambda i: (0, i, 0))


def _full_spec(shape):
    return pl.BlockSpec(shape, lambda i: tuple(0 for _ in shape))


def kernel(x, edge_index, W1, b1, W2, b2):
    src = edge_index[0].astype(jnp.int32)
    dst = edge_index[1].astype(jnp.int32)
    pad = EPAD - E
    srcp = jnp.concatenate([src, jnp.zeros((pad,), jnp.int32)])
    dstp = jnp.concatenate([dst, jnp.full((pad,), TRASH, jnp.int32)])
    dst_w = dstp.reshape(NC * NS, EPW)
    src3 = srcp.reshape(NS, K, CH)
    dst3 = dstp.reshape(NS, K, CH)
    b1r = b1.reshape(1, D)
    b2r = b2.reshape(1, D)

    deg_kernel, scatter_kernel = _sc_kernels()
    deg_parts = deg_kernel(dst_w).T

    g1 = pl.pallas_call(
        _mm_scale_body,
        grid=(GRID,),
        in_specs=[_deg_spec(), _row_spec(), _full_spec((D, D))],
        out_specs=_row_spec(),
        out_shape=jax.ShapeDtypeStruct((N, D), jnp.float32),
    )(deg_parts, x, W1)

    s1 = scatter_kernel(g1.reshape(4 * N, DQ), src3, dst3)

    g2 = pl.pallas_call(
        _combine_mm_body,
        grid=(GRID,),
        in_specs=[_deg_spec(), _sp_spec(), _row_spec(),
                  _full_spec((1, D)), _full_spec((D, D))],
        out_specs=_row_spec(),
        out_shape=jax.ShapeDtypeStruct((N, D), jnp.float32),
    )(deg_parts, s1, g1, b1r, W2)

    s2 = scatter_kernel(g2.reshape(4 * N, DQ), src3, dst3)

    o2, stats = pl.pallas_call(
        _combine_stats_body,
        grid=(GRID,),
        in_specs=[_deg_spec(), _sp_spec(), _row_spec(), _full_spec((1, D))],
        out_specs=[_row_spec(), _full_spec((2, D))],
        out_shape=[jax.ShapeDtypeStruct((N, D), jnp.float32),
                   jax.ShapeDtypeStruct((2, D), jnp.float32)],
    )(deg_parts, s2, g2, b2r)

    out = pl.pallas_call(
        _norm_body,
        grid=(GRID,),
        in_specs=[_row_spec(), _full_spec((2, D))],
        out_specs=_row_spec(),
        out_shape=jax.ShapeDtypeStruct((N, D), jnp.float32),
    )(o2, stats)
    return out


# speedup vs baseline: 1.2663x; 1.1129x over previous
import functools

import jax
import jax.numpy as jnp
from jax import lax
from jax.experimental import pallas as pl
from jax.experimental.pallas import tpu as pltpu
from jax.experimental.pallas import tpu_sc as plsc

N = 10000
E = 160000
D = 256
DH = 128
DQ = 64
NC = 2
NS = 16
CH = 128
K = 80
NB = 4
EPAD = NS * K * CH
EPW = EPAD // (NC * NS)
ROWS_ACC = 10240
TRASH = N
RPT = ROWS_ACC // NS
RB = 1000
GRID = N // RB

@functools.lru_cache(maxsize=1)
def _sc_kernels():
    mesh = plsc.VectorSubcoreMesh(
        core_axis_name="c", subcore_axis_name="s",
        num_cores=NC, num_subcores=NS)

    deg_kernel = functools.partial(
        pl.kernel,
        out_type=jax.ShapeDtypeStruct((NC * NS, ROWS_ACC), jnp.float32),
        mesh=mesh,
        scratch_types=[
            pltpu.VMEM((EPW,), jnp.int32),
            pltpu.VMEM((ROWS_ACC,), jnp.float32),
        ],
        compiler_params=pltpu.CompilerParams(
            needs_layout_passes=False, use_tc_tiling_on_sc=False),
    )(_deg_body)

    scatter_kernel = functools.partial(
        pl.kernel,
        out_type=jax.ShapeDtypeStruct((2 * NC, ROWS_ACC, DQ), jnp.float32),
        mesh=mesh,
        scratch_types=[
            pltpu.VMEM((K, CH), jnp.int32),
            pltpu.VMEM((K, CH), jnp.int32),
            pltpu.VMEM((K, CH), jnp.int32),
            *[pltpu.VMEM((CH, DQ // 2), jnp.int32) for _ in range(NB)],
            pltpu.VMEM((CH, DQ), jnp.float32),
            pltpu.VMEM((CH, DQ), jnp.float32),
            pltpu.VMEM((CH, DQ), jnp.float32),
            pltpu.VMEM_SHARED((ROWS_ACC, DQ), jnp.float32),
            *[pltpu.SemaphoreType.DMA for _ in range(NB)],
            pltpu.SemaphoreType.DMA,
            pltpu.SemaphoreType.DMA,
        ],
        compiler_params=pltpu.CompilerParams(
            needs_layout_passes=False, use_tc_tiling_on_sc=False),
    )(_scatter_body)
    return deg_kernel, scatter_kernel


def _deg_body(dst_hbm, out_hbm, dstv, hist):
    cid = lax.axis_index("c")
    sid = lax.axis_index("s")
    wid = sid * NC + cid

    def zb(i, carry):
        hist[pl.ds(i * 16, 16)] = jnp.zeros((16,), jnp.float32)
        return carry

    lax.fori_loop(0, ROWS_ACC // 16, zb, 0)
    pltpu.sync_copy(dst_hbm.at[wid], dstv)
    ones = jnp.ones((16,), jnp.float32)

    def body(j, carry):
        idx = dstv[pl.ds(j * 16, 16)]
        plsc.addupdate_scatter(hist, [idx], ones)
        return carry

    lax.fori_loop(0, EPW // 16, body, 0)
    pltpu.sync_copy(hist, out_hbm.at[wid])


def _scatter_body(g_hbm, src_hbm, dst_hbm, out_hbm,
                  srcv, gidx, didx, *rest):
    rows = rest[:NB]
    f32b = rest[NB:NB + 2]
    zbuf, acc = rest[NB + 2], rest[NB + 3]
    gsem = rest[NB + 4:NB + 4 + NB]
    ssem = rest[NB + 4 + NB:]
    cid = lax.axis_index("c")
    sid = lax.axis_index("s")

    def zb(i, carry):
        r = i // (DQ // 16)
        c = (i % (DQ // 16)) * 16
        zbuf[r, pl.ds(c, 16)] = jnp.zeros((16,), jnp.float32)
        return carry

    lax.fori_loop(0, CH * (DQ // 16), zb, 0)

    pltpu.sync_copy(src_hbm.at[sid], srcv)
    pltpu.sync_copy(dst_hbm.at[sid], didx)

    for qpass in range(2):
        q = cid * 2 + qpass
        for k in range(RPT // CH):
            pltpu.sync_copy(zbuf, acc.at[pl.ds(sid * RPT + k * CH, CH)])

        def tb(i, carry):
            r = i // (CH // 16)
            c = (i % (CH // 16)) * 16
            sv = srcv[r, pl.ds(c, 16)]
            gidx[r, pl.ds(c, 16)] = sv * 4 + q
            return carry

        lax.fori_loop(0, K * (CH // 16), tb, 0)
        plsc.subcore_barrier()

        def _gather(j, b):
            pltpu.async_copy(g_hbm.at[gidx.at[j]], rows[b], gsem[b])

        def _scatter(j, p):
            pltpu.async_copy(f32b[p], acc.at[didx.at[j]], ssem[p], add=True)

        def _gwait(b):
            pltpu.make_async_copy(g_hbm.at[gidx.at[0]], rows[b],
                                  gsem[b]).wait()

        def _swait(p):
            pltpu.make_async_copy(f32b[p], acc.at[didx.at[0]],
                                  ssem[p]).wait()

        def convert(b, p):
            def cv(i, carry):
                r = i // (DQ // 32)
                t = i % (DQ // 32)
                w = rows[b][r, pl.ds(t * 16, 16)]
                lo = plsc.bitcast(lax.shift_left(w, 16), jnp.float32)
                hi = plsc.bitcast(w & jnp.int32(-65536), jnp.float32)
                f32b[p][r, pl.ds(t * 16, 16)] = lo
                f32b[p][r, pl.ds(32 + t * 16, 16)] = hi
                return carry

            lax.fori_loop(0, CH * (DQ // 32), cv, 0)

        for b in range(NB):
            _gather(b, b)

        def grp(g, carry):
            for b in range(NB):
                jj = (g - 1) * NB + b
                p = b % 2
                _gwait(b)

                @pl.when(jj >= 2)
                def _():
                    _swait(p)

                convert(b, p)
                _scatter(jj, p)
                _gather(g * NB + b, b)
            return carry

        lax.fori_loop(1, K // NB, grp, 0)
        for b in range(NB):
            jj = K - NB + b
            p = b % 2
            _gwait(b)
            _swait(p)
            convert(b, p)
            _scatter(jj, p)
        _swait(0)
        _swait(1)
        plsc.subcore_barrier()
        pltpu.sync_copy(acc.at[pl.ds(sid * RPT, RPT)],
                        out_hbm.at[q, pl.ds(sid * RPT, RPT)])


def _pack_rows(h):
    u = lax.bitcast_convert_type(h, jnp.int32)
    r = u + jnp.int32(0x7FFF) + ((u >> 16) & 1)
    top = lax.shift_right_logical(r, 16)
    return top[:, :D // 2] | (top[:, D // 2:] << 16)


def _psi_assemble(sp, gp):
    g_lo = lax.bitcast_convert_type(gp << 16, jnp.float32)
    g_hi = lax.bitcast_convert_type(gp & jnp.int32(-65536), jnp.float32)
    s = jnp.concatenate([sp[0], sp[1], sp[2], sp[3]], axis=-1)
    g = jnp.concatenate(
        [x for q in range(4)
         for x in (g_lo[:, 32 * q:32 * q + 32], g_hi[:, 32 * q:32 * q + 32])],
        axis=-1)
    return s, g


def _psi_to_natural(o):
    return jnp.concatenate(
        [o[:, 64 * q:64 * q + 32] for q in range(4)]
        + [o[:, 64 * q + 32:64 * q + 64] for q in range(4)], axis=-1)


def _mm_scale_body(deg_ref, x_ref, w_ref, out_ref):
    deg = jnp.sum(deg_ref[...], axis=1) + 1.0
    dinv = lax.rsqrt(deg)
    h = jnp.dot(x_ref[...], w_ref[...], preferred_element_type=jnp.float32)
    out_ref[...] = _pack_rows(h * dinv[:, None])


def _combine_mm_body(deg_ref, sp_ref, g_ref, b_ref, w_ref, out_ref):
    deg = jnp.sum(deg_ref[...], axis=1) + 1.0
    dinv = lax.rsqrt(deg)
    s, g = _psi_assemble(sp_ref[...], g_ref[...])
    o = (s + g) * dinv[:, None] + b_ref[...]
    h = jnp.dot(o, w_ref[...], preferred_element_type=jnp.float32)
    out_ref[...] = _pack_rows(h * dinv[:, None])


def _combine_stats_body(deg_ref, sp_ref, g_ref, b_ref, o_ref, stats_ref):
    i = pl.program_id(0)
    deg = jnp.sum(deg_ref[...], axis=1) + 1.0
    dinv = lax.rsqrt(deg)
    s, g = _psi_assemble(sp_ref[...], g_ref[...])
    o = _psi_to_natural((s + g) * dinv[:, None] + b_ref[...])
    o_ref[...] = o
    blk = jnp.stack([jnp.sum(o, axis=0), jnp.sum(o * o, axis=0)])

    @pl.when(i == 0)
    def _():
        stats_ref[...] = blk

    @pl.when(i > 0)
    def _():
        stats_ref[...] = stats_ref[...] + blk


def _norm_body(o_ref, stats_ref, out_ref):
    st = stats_ref[...]
    mean = st[0]
    nf = jnp.float32(N)
    mu = mean / nf
    var = (st[1] - nf * mu * mu) / (nf - 1.0)
    rstd = lax.rsqrt(var)
    out_ref[...] = (o_ref[...] - mu[None, :]) * rstd[None, :]


def _deg_spec():
    return pl.BlockSpec((RB, NC * NS), lambda i: (i, 0))


def _row_spec():
    return pl.BlockSpec((RB, D), lambda i: (i, 0))


def _packed_spec():
    return pl.BlockSpec((RB, D // 2), lambda i: (i, 0))


def _sp_spec():
    return pl.BlockSpec((2 * NC, RB, DQ), lambda i: (0, i, 0))


def _full_spec(shape):
    return pl.BlockSpec(shape, lambda i: tuple(0 for _ in shape))


def kernel(x, edge_index, W1, b1, W2, b2):
    src = edge_index[0].astype(jnp.int32)
    dst = edge_index[1].astype(jnp.int32)
    pad = EPAD - E
    srcp = jnp.concatenate([src, jnp.zeros((pad,), jnp.int32)])
    dstp = jnp.concatenate([dst, jnp.full((pad,), TRASH, jnp.int32)])
    dst_w = dstp.reshape(NC * NS, EPW)
    src3 = srcp.reshape(NS, K, CH)
    dst3 = dstp.reshape(NS, K, CH)
    psi = jnp.asarray(
        [f for q in range(4)
         for f in list(range(32 * q, 32 * q + 32))
         + list(range(128 + 32 * q, 128 + 32 * q + 32))], dtype=jnp.int32)
    b1r = b1[psi].reshape(1, D)
    b2r = b2[psi].reshape(1, D)
    W2p = W2[psi, :]

    deg_kernel, scatter_kernel = _sc_kernels()
    deg_parts = deg_kernel(dst_w).T

    g1 = pl.pallas_call(
        _mm_scale_body,
        grid=(GRID,),
        in_specs=[_deg_spec(), _row_spec(), _full_spec((D, D))],
        out_specs=_packed_spec(),
        out_shape=jax.ShapeDtypeStruct((N, D // 2), jnp.int32),
    )(deg_parts, x, W1)

    s1 = scatter_kernel(g1.reshape(4 * N, DQ // 2), src3, dst3)

    g2 = pl.pallas_call(
        _combine_mm_body,
        grid=(GRID,),
        in_specs=[_deg_spec(), _sp_spec(), _packed_spec(),
                  _full_spec((1, D)), _full_spec((D, D))],
        out_specs=_packed_spec(),
        out_shape=jax.ShapeDtypeStruct((N, D // 2), jnp.int32),
    )(deg_parts, s1, g1, b1r, W2p)

    s2 = scatter_kernel(g2.reshape(4 * N, DQ // 2), src3, dst3)

    o2, stats = pl.pallas_call(
        _combine_stats_body,
        grid=(GRID,),
        in_specs=[_deg_spec(), _sp_spec(), _packed_spec(),
                  _full_spec((1, D))],
        out_specs=[_row_spec(), _full_spec((2, D))],
        out_shape=[jax.ShapeDtypeStruct((N, D), jnp.float32),
                   jax.ShapeDtypeStruct((2, D), jnp.float32)],
    )(deg_parts, s2, g2, b2r)

    out = pl.pallas_call(
        _norm_body,
        grid=(GRID,),
        in_specs=[_row_spec(), _full_spec((2, D))],
        out_specs=_row_spec(),
        out_shape=jax.ShapeDtypeStruct((N, D), jnp.float32),
    )(o2, stats)
    return out


# speedup vs baseline: 1.2663x; 1.0000x over previous
import functools

import jax
import jax.numpy as jnp
from jax import lax
from jax.experimental import pallas as pl
from jax.experimental.pallas import tpu as pltpu
from jax.experimental.pallas import tpu_sc as plsc

N = 10000
E = 160000
D = 256
DH = 128
DQ = 64
NC = 2
NS = 16
CH = 128
K = 80
NB = 4
EPAD = NS * K * CH
EPW = EPAD // (NC * NS)
ROWS_ACC = 10240
TRASH = N
RPT = ROWS_ACC // NS
RB = 1000
GRID = N // RB

@functools.lru_cache(maxsize=1)
def _sc_kernels():
    mesh = plsc.VectorSubcoreMesh(
        core_axis_name="c", subcore_axis_name="s",
        num_cores=NC, num_subcores=NS)

    deg_kernel = functools.partial(
        pl.kernel,
        out_type=jax.ShapeDtypeStruct((NC * NS, ROWS_ACC), jnp.float32),
        mesh=mesh,
        scratch_types=[
            pltpu.VMEM((EPW,), jnp.int32),
            pltpu.VMEM((ROWS_ACC,), jnp.float32),
        ],
        compiler_params=pltpu.CompilerParams(
            needs_layout_passes=False, use_tc_tiling_on_sc=False),
    )(_deg_body)

    scatter_kernel = functools.partial(
        pl.kernel,
        out_type=jax.ShapeDtypeStruct((2 * NC, ROWS_ACC, DQ), jnp.float32),
        mesh=mesh,
        scratch_types=[
            pltpu.VMEM((K, CH), jnp.int32),
            pltpu.VMEM((K, CH), jnp.int32),
            pltpu.VMEM((K, CH), jnp.int32),
            *[pltpu.VMEM((CH, DQ // 2), jnp.int32) for _ in range(NB)],
            *[pltpu.VMEM((CH, DQ), jnp.float32) for _ in range(NB)],
            pltpu.VMEM((CH, DQ), jnp.float32),
            pltpu.VMEM_SHARED((ROWS_ACC, DQ), jnp.float32),
            *[pltpu.SemaphoreType.DMA for _ in range(2 * NB)],
        ],
        compiler_params=pltpu.CompilerParams(
            needs_layout_passes=False, use_tc_tiling_on_sc=False),
    )(_scatter_body)
    return deg_kernel, scatter_kernel


def _deg_body(dst_hbm, out_hbm, dstv, hist):
    cid = lax.axis_index("c")
    sid = lax.axis_index("s")
    wid = sid * NC + cid

    def zb(i, carry):
        hist[pl.ds(i * 16, 16)] = jnp.zeros((16,), jnp.float32)
        return carry

    lax.fori_loop(0, ROWS_ACC // 16, zb, 0)
    pltpu.sync_copy(dst_hbm.at[wid], dstv)
    ones = jnp.ones((16,), jnp.float32)

    def body(j, carry):
        idx = dstv[pl.ds(j * 16, 16)]
        plsc.addupdate_scatter(hist, [idx], ones)
        return carry

    lax.fori_loop(0, EPW // 16, body, 0)
    pltpu.sync_copy(hist, out_hbm.at[wid])


def _scatter_body(g_hbm, src_hbm, dst_hbm, out_hbm,
                  srcv, gidx, didx, *rest):
    rows = rest[:NB]
    f32b = rest[NB:2 * NB]
    zbuf, acc = rest[2 * NB], rest[2 * NB + 1]
    gsem = rest[2 * NB + 2:2 * NB + 2 + NB]
    ssem = rest[2 * NB + 2 + NB:]
    cid = lax.axis_index("c")
    sid = lax.axis_index("s")

    def zb(i, carry):
        r = i // (DQ // 16)
        c = (i % (DQ // 16)) * 16
        zbuf[r, pl.ds(c, 16)] = jnp.zeros((16,), jnp.float32)
        return carry

    lax.fori_loop(0, CH * (DQ // 16), zb, 0)

    pltpu.sync_copy(src_hbm.at[sid], srcv)
    pltpu.sync_copy(dst_hbm.at[sid], didx)

    for qpass in range(2):
        q = cid * 2 + qpass
        for k in range(RPT // CH):
            pltpu.sync_copy(zbuf, acc.at[pl.ds(sid * RPT + k * CH, CH)])

        def tb(i, carry):
            r = i // (CH // 16)
            c = (i % (CH // 16)) * 16
            sv = srcv[r, pl.ds(c, 16)]
            gidx[r, pl.ds(c, 16)] = sv * 4 + q
            return carry

        lax.fori_loop(0, K * (CH // 16), tb, 0)
        plsc.subcore_barrier()

        def _gather(j, b):
            pltpu.async_copy(g_hbm.at[gidx.at[j]], rows[b], gsem[b])

        def _scatter(j, p):
            pltpu.async_copy(f32b[p], acc.at[didx.at[j]], ssem[p], add=True)

        def _gwait(b):
            pltpu.make_async_copy(g_hbm.at[gidx.at[0]], rows[b],
                                  gsem[b]).wait()

        def _swait(p):
            pltpu.make_async_copy(f32b[p], acc.at[didx.at[0]],
                                  ssem[p]).wait()

        def convert(b, p):
            def cv(i, carry):
                r = i // (DQ // 32)
                t = i % (DQ // 32)
                w = rows[b][r, pl.ds(t * 16, 16)]
                lo = plsc.bitcast(lax.shift_left(w, 16), jnp.float32)
                hi = plsc.bitcast(w & jnp.int32(-65536), jnp.float32)
                f32b[p][r, pl.ds(t * 16, 16)] = lo
                f32b[p][r, pl.ds(32 + t * 16, 16)] = hi
                return carry

            lax.fori_loop(0, CH * (DQ // 32), cv, 0)

        for b in range(NB):
            _gather(b, b)

        def grp(g, carry):
            for b in range(NB):
                jj = (g - 1) * NB + b
                _gwait(b)

                @pl.when(jj >= NB)
                def _():
                    _swait(b)

                convert(b, b)
                _scatter(jj, b)
                _gather(g * NB + b, b)
            return carry

        lax.fori_loop(1, K // NB, grp, 0)
        for b in range(NB):
            jj = K - NB + b
            _gwait(b)
            _swait(b)
            convert(b, b)
            _scatter(jj, b)
        for b in range(NB):
            _swait(b)
        plsc.subcore_barrier()
        pltpu.sync_copy(acc.at[pl.ds(sid * RPT, RPT)],
                        out_hbm.at[q, pl.ds(sid * RPT, RPT)])


def _pack_rows(h):
    u = lax.bitcast_convert_type(h, jnp.int32)
    r = u + jnp.int32(0x7FFF) + ((u >> 16) & 1)
    top = lax.shift_right_logical(r, 16)
    return top[:, :D // 2] | (top[:, D // 2:] << 16)


def _psi_assemble(sp, gp):
    g_lo = lax.bitcast_convert_type(gp << 16, jnp.float32)
    g_hi = lax.bitcast_convert_type(gp & jnp.int32(-65536), jnp.float32)
    s = jnp.concatenate([sp[0], sp[1], sp[2], sp[3]], axis=-1)
    g = jnp.concatenate(
        [x for q in range(4)
         for x in (g_lo[:, 32 * q:32 * q + 32], g_hi[:, 32 * q:32 * q + 32])],
        axis=-1)
    return s, g


def _psi_to_natural(o):
    return jnp.concatenate(
        [o[:, 64 * q:64 * q + 32] for q in range(4)]
        + [o[:, 64 * q + 32:64 * q + 64] for q in range(4)], axis=-1)


def _mm_scale_body(deg_ref, x_ref, w_ref, out_ref):
    deg = jnp.sum(deg_ref[...], axis=1) + 1.0
    dinv = lax.rsqrt(deg)
    h = jnp.dot(x_ref[...], w_ref[...], preferred_element_type=jnp.float32)
    out_ref[...] = _pack_rows(h * dinv[:, None])


def _combine_mm_body(deg_ref, sp_ref, g_ref, b_ref, w_ref, out_ref):
    deg = jnp.sum(deg_ref[...], axis=1) + 1.0
    dinv = lax.rsqrt(deg)
    s, g = _psi_assemble(sp_ref[...], g_ref[...])
    o = (s + g) * dinv[:, None] + b_ref[...]
    h = jnp.dot(o, w_ref[...], preferred_element_type=jnp.float32)
    out_ref[...] = _pack_rows(h * dinv[:, None])


def _combine_stats_body(deg_ref, sp_ref, g_ref, b_ref, o_ref, stats_ref):
    i = pl.program_id(0)
    deg = jnp.sum(deg_ref[...], axis=1) + 1.0
    dinv = lax.rsqrt(deg)
    s, g = _psi_assemble(sp_ref[...], g_ref[...])
    o = _psi_to_natural((s + g) * dinv[:, None] + b_ref[...])
    o_ref[...] = o
    blk = jnp.stack([jnp.sum(o, axis=0), jnp.sum(o * o, axis=0)])

    @pl.when(i == 0)
    def _():
        stats_ref[...] = blk

    @pl.when(i > 0)
    def _():
        stats_ref[...] = stats_ref[...] + blk


def _norm_body(o_ref, stats_ref, out_ref):
    st = stats_ref[...]
    mean = st[0]
    nf = jnp.float32(N)
    mu = mean / nf
    var = (st[1] - nf * mu * mu) / (nf - 1.0)
    rstd = lax.rsqrt(var)
    out_ref[...] = (o_ref[...] - mu[None, :]) * rstd[None, :]


def _deg_spec():
    return pl.BlockSpec((RB, NC * NS), lambda i: (i, 0))


def _row_spec():
    return pl.BlockSpec((RB, D), lambda i: (i, 0))


def _packed_spec():
    return pl.BlockSpec((RB, D // 2), lambda i: (i, 0))


def _sp_spec():
    return pl.BlockSpec((2 * NC, RB, DQ), lambda i: (0, i, 0))


def _full_spec(shape):
    return pl.BlockSpec(shape, lambda i: tuple(0 for _ in shape))


def kernel(x, edge_index, W1, b1, W2, b2):
    src = edge_index[0].astype(jnp.int32)
    dst = edge_index[1].astype(jnp.int32)
    pad = EPAD - E
    srcp = jnp.concatenate([src, jnp.zeros((pad,), jnp.int32)])
    dstp = jnp.concatenate([dst, jnp.full((pad,), TRASH, jnp.int32)])
    dst_w = dstp.reshape(NC * NS, EPW)
    src3 = srcp.reshape(NS, K, CH)
    dst3 = dstp.reshape(NS, K, CH)
    psi = jnp.asarray(
        [f for q in range(4)
         for f in list(range(32 * q, 32 * q + 32))
         + list(range(128 + 32 * q, 128 + 32 * q + 32))], dtype=jnp.int32)
    b1r = b1[psi].reshape(1, D)
    b2r = b2[psi].reshape(1, D)
    W2p = W2[psi, :]

    deg_kernel, scatter_kernel = _sc_kernels()
    deg_parts = deg_kernel(dst_w).T

    g1 = pl.pallas_call(
        _mm_scale_body,
        grid=(GRID,),
        in_specs=[_deg_spec(), _row_spec(), _full_spec((D, D))],
        out_specs=_packed_spec(),
        out_shape=jax.ShapeDtypeStruct((N, D // 2), jnp.int32),
    )(deg_parts, x, W1)

    s1 = scatter_kernel(g1.reshape(4 * N, DQ // 2), src3, dst3)

    g2 = pl.pallas_call(
        _combine_mm_body,
        grid=(GRID,),
        in_specs=[_deg_spec(), _sp_spec(), _packed_spec(),
                  _full_spec((1, D)), _full_spec((D, D))],
        out_specs=_packed_spec(),
        out_shape=jax.ShapeDtypeStruct((N, D // 2), jnp.int32),
    )(deg_parts, s1, g1, b1r, W2p)

    s2 = scatter_kernel(g2.reshape(4 * N, DQ // 2), src3, dst3)

    o2, stats = pl.pallas_call(
        _combine_stats_body,
        grid=(GRID,),
        in_specs=[_deg_spec(), _sp_spec(), _packed_spec(),
                  _full_spec((1, D))],
        out_specs=[_row_spec(), _full_spec((2, D))],
        out_shape=[jax.ShapeDtypeStruct((N, D), jnp.float32),
                   jax.ShapeDtypeStruct((2, D), jnp.float32)],
    )(deg_parts, s2, g2, b2r)

    out = pl.pallas_call(
        _norm_body,
        grid=(GRID,),
        in_specs=[_row_spec(), _full_spec((2, D))],
        out_specs=_row_spec(),
        out_shape=jax.ShapeDtypeStruct((N, D), jnp.float32),
    )(o2, stats)
    return out
